# baseline (speedup 1.0000x reference)
import functools

import jax
import jax.numpy as jnp
from jax import lax
from jax.experimental import pallas as pl
from jax.experimental.pallas import tpu as pltpu
from jax.experimental.pallas import tpu_sc as plsc

N = 10000
D = 128
K = 16
NW = 32
CH = 128
NBUF = 4


def _tc_body(z_ref, c_ref, psi_ref, t_ref):
    w = lax.dot_general(z_ref[...], psi_ref[...], (((1,), (1,)), ((), ())),
                        preferred_element_type=jnp.float32,
                        precision=lax.Precision.HIGHEST)
    t_ref[pl.ds(0, N), :] = w
    t_ref[pl.ds(N, N), :] = c_ref[...]


def _build_table(z, c, psi):
    return pl.pallas_call(
        _tc_body,
        out_shape=jax.ShapeDtypeStruct((2 * N, K), jnp.float32),
    )(z, c, psi)


def _sigmoid(x):
    return 1.0 / (1.0 + jnp.exp(-x))


def _sc_edge_scores(nch):
    mesh = plsc.VectorSubcoreMesh(core_axis_name="c", subcore_axis_name="s")
    e_pad = NW * nch * CH

    @functools.partial(
        pl.kernel,
        out_type=jax.ShapeDtypeStruct((e_pad,), jnp.float32),
        mesh=mesh,
        compiler_params=pltpu.CompilerParams(
            needs_layout_passes=False, use_tc_tiling_on_sc=False),
        scratch_types=[
            pltpu.VMEM((4 * nch, CH), jnp.int32),
            pltpu.VMEM((NBUF, CH, K), jnp.float32),
            pltpu.VMEM((NBUF, CH, K), jnp.float32),
            pltpu.VMEM((NBUF, CH, K), jnp.float32),
            pltpu.VMEM((NBUF, CH, K), jnp.float32),
            pltpu.VMEM((nch * CH,), jnp.float32),
            pltpu.VMEM_SHARED((2 * N, K), jnp.float32),
            pltpu.SemaphoreType.DMA,
            pltpu.SemaphoreType.DMA,
            pltpu.SemaphoreType.DMA,
            pltpu.SemaphoreType.DMA,
        ],
    )
    def k(t_hbm, idx_hbm, out_hbm,
          idx_v, wrr, crr, wcr, ccr, obf, spt, sem0, sem1, sem2, sem3):
        nc = plsc.get_sparse_core_info().num_cores
        wid = lax.axis_index("s") * nc + lax.axis_index("c")
        crow = wid * nch

        @pl.when(lax.axis_index("s") == 0)
        def _():
            pltpu.sync_copy(t_hbm, spt)
        pltpu.sync_copy(idx_hbm.at[pl.ds(4 * crow, 4 * nch)], idx_v)
        plsc.subcore_barrier()

        sems = (sem0, sem1, sem2, sem3)

        def start(i, b):
            semb = sems[b]
            pltpu.async_copy(spt.at[idx_v.at[4 * i]], wrr.at[b], semb)
            pltpu.async_copy(spt.at[idx_v.at[4 * i + 1]], crr.at[b], semb)
            pltpu.async_copy(spt.at[idx_v.at[4 * i + 2]], wcr.at[b], semb)
            pltpu.async_copy(spt.at[idx_v.at[4 * i + 3]], ccr.at[b], semb)

        def wait(b):
            semb = sems[b]
            for buf in (wrr, crr, wcr, ccr):
                pltpu.make_async_copy(
                    t_hbm.at[pl.ds(0, CH)], buf.at[b], semb).wait()

        lane = lax.iota(jnp.int32, 16)

        def compute(i, b):
            def group(t, _):
                za = jnp.zeros((16,), jnp.float32)
                r1 = za
                r2 = za
                for j in range(16):
                    e = t * 16 + j
                    wr = wrr[b, e, pl.ds(0, K)]
                    cr = crr[b, e, pl.ds(0, K)]
                    wc = wcr[b, e, pl.ds(0, K)]
                    cc = ccr[b, e, pl.ds(0, K)]
                    s1 = jnp.sum(wr * cc)
                    s2 = jnp.sum(wc * cr)
                    m = lane == j
                    r1 = jnp.where(m, jnp.full((16,), s1), r1)
                    r2 = jnp.where(m, jnp.full((16,), s2), r2)
                s = 0.5 * (_sigmoid(r1) + _sigmoid(r2))
                obf[pl.ds(i * CH + t * 16, 16)] = s
                return 0

            lax.fori_loop(0, CH // 16, group, 0)

        for b in range(NBUF):
            start(b, b)

        def ring(p, _):
            for b in range(NBUF):
                i = p * NBUF + b
                wait(b)
                compute(i, b)

                @pl.when(i + NBUF < nch)
                def _():
                    start(i + NBUF, b)
            return 0

        lax.fori_loop(0, nch // NBUF, ring, 0)
        pltpu.sync_copy(obf, out_hbm.at[pl.ds(crow * CH, nch * CH)])

    return k


def kernel(z, edge_index, c, psi, mp_samples):
    del mp_samples
    e = edge_index.shape[1]
    nch = -(-e // (NW * CH * NBUF)) * NBUF
    e_pad = NW * nch * CH
    row = jnp.pad(edge_index[0].astype(jnp.int32), (0, e_pad - e))
    col = jnp.pad(edge_index[1].astype(jnp.int32), (0, e_pad - e))
    rowm = row.reshape(NW * nch, 1, CH)
    colm = col.reshape(NW * nch, 1, CH)
    idx4 = jnp.concatenate(
        [rowm, rowm + N, colm, colm + N], axis=1).reshape(4 * NW * nch, CH)
    t = _build_table(z, c, psi)
    out = _sc_edge_scores(nch)(t, idx4, )
    return out[:e]


# speedup vs baseline: 1.0605x; 1.0605x over previous
import functools

import jax
import jax.numpy as jnp
from jax import lax
from jax.experimental import pallas as pl
from jax.experimental.pallas import tpu as pltpu
from jax.experimental.pallas import tpu_sc as plsc

N = 10000
D = 128
K = 16
NW = 32
CH = 128
NBUF = 2


def _tc_body(z_ref, psi_ref, w_ref):
    w_ref[...] = lax.dot_general(
        z_ref[...], psi_ref[...], (((1,), (1,)), ((), ())),
        preferred_element_type=jnp.float32,
        precision=lax.Precision.HIGHEST)


def _build_w(z, psi):
    return pl.pallas_call(
        _tc_body,
        out_shape=jax.ShapeDtypeStruct((N, K), jnp.float32),
    )(z, psi)


def _sigmoid(x):
    return 1.0 / (1.0 + jnp.exp(-x))


def _sc_edge_scores(nch):
    mesh = plsc.VectorSubcoreMesh(core_axis_name="c", subcore_axis_name="s")
    e_pad = NW * nch * CH

    @functools.partial(
        pl.kernel,
        out_type=jax.ShapeDtypeStruct((e_pad,), jnp.float32),
        mesh=mesh,
        compiler_params=pltpu.CompilerParams(
            needs_layout_passes=False, use_tc_tiling_on_sc=False),
        scratch_types=[
            pltpu.VMEM((nch, CH), jnp.int32),
            pltpu.VMEM((nch, CH), jnp.int32),
            pltpu.VMEM((CH, 2 * K), jnp.float32),
            pltpu.VMEM((CH, 2 * K), jnp.float32),
            pltpu.VMEM((CH, 2 * K), jnp.float32),
            pltpu.VMEM((CH, 2 * K), jnp.float32),
            pltpu.VMEM((nch * CH,), jnp.float32),
            pltpu.VMEM_SHARED((N, 2 * K), jnp.float32),
            pltpu.SemaphoreType.DMA,
            pltpu.SemaphoreType.DMA,
        ],
    )
    def k(w_hbm, c_hbm, row_hbm, col_hbm, out_hbm,
          idx_r, idx_c, gr0, gr1, gc0, gc1, obf, spg, sem0, sem1):
        nc = plsc.get_sparse_core_info().num_cores
        sid = lax.axis_index("s")
        wid = sid * nc + lax.axis_index("c")
        crow = wid * nch

        @pl.when(sid == 0)
        def _():
            pltpu.sync_copy(w_hbm, spg.at[:, pl.ds(0, K)])

        @pl.when(sid == 1)
        def _():
            pltpu.sync_copy(c_hbm, spg.at[:, pl.ds(K, K)])

        pltpu.sync_copy(row_hbm.at[pl.ds(crow, nch)], idx_r)
        pltpu.sync_copy(col_hbm.at[pl.ds(crow, nch)], idx_c)
        plsc.subcore_barrier()

        slots = ((gr0, gc0, sem0), (gr1, gc1, sem1))

        def start(i, b):
            grb, gcb, semb = slots[b]
            pltpu.async_copy(spg.at[idx_r.at[i]], grb, semb)
            pltpu.async_copy(spg.at[idx_c.at[i]], gcb, semb)

        def wait(i, b):
            grb, gcb, semb = slots[b]
            pltpu.make_async_copy(spg.at[idx_r.at[i]], grb, semb).wait()
            pltpu.make_async_copy(spg.at[idx_c.at[i]], gcb, semb).wait()

        lane = lax.iota(jnp.int32, 16)

        def compute(i, b):
            grb, gcb, _ = slots[b]

            def group(t, _):
                za = jnp.zeros((16,), jnp.float32)
                r1 = za
                r2 = za
                for j in range(16):
                    e = t * 16 + j
                    wr = grb[e, pl.ds(0, K)]
                    cr = grb[e, pl.ds(K, K)]
                    wc = gcb[e, pl.ds(0, K)]
                    cc = gcb[e, pl.ds(K, K)]
                    s1 = jnp.sum(wr * cc)
                    s2 = jnp.sum(wc * cr)
                    m = lane == j
                    r1 = jnp.where(m, jnp.full((16,), s1), r1)
                    r2 = jnp.where(m, jnp.full((16,), s2), r2)
                s = 0.5 * (_sigmoid(r1) + _sigmoid(r2))
                obf[pl.ds(i * CH + t * 16, 16)] = s
                return 0

            lax.fori_loop(0, CH // 16, group, 0)

        for b in range(NBUF):
            start(b, b)

        def ring(p, _):
            for b in range(NBUF):
                i = p * NBUF + b
                wait(i, b)
                compute(i, b)

                @pl.when(i + NBUF < nch)
                def _():
                    start(i + NBUF, b)
            return 0

        lax.fori_loop(0, nch // NBUF, ring, 0)
        pltpu.sync_copy(obf, out_hbm.at[pl.ds(crow * CH, nch * CH)])

    return k


def kernel(z, edge_index, c, psi, mp_samples):
    del mp_samples
    e = edge_index.shape[1]
    nch = -(-e // (NW * CH * NBUF)) * NBUF
    e_pad = NW * nch * CH
    row = jnp.pad(edge_index[0].astype(jnp.int32), (0, e_pad - e))
    col = jnp.pad(edge_index[1].astype(jnp.int32), (0, e_pad - e))
    row = row.reshape(NW * nch, CH)
    col = col.reshape(NW * nch, CH)
    w = _build_w(z, psi)
    out = _sc_edge_scores(nch)(w, c, row, col)
    return out[:e]


# speedup vs baseline: 1.1380x; 1.0732x over previous
import functools

import jax
import jax.numpy as jnp
from jax import lax
from jax.experimental import pallas as pl
from jax.experimental.pallas import tpu as pltpu
from jax.experimental.pallas import tpu_sc as plsc

N = 10000
D = 128
K = 16
NW = 32
CH = 128
NBUF = 2


def _tc_body(z_ref, c_ref, psi_ref, g_ref):
    w = lax.dot_general(z_ref[...], psi_ref[...], (((1,), (1,)), ((), ())),
                        preferred_element_type=jnp.float32,
                        precision=lax.Precision.HIGHEST)
    g_ref[...] = jnp.concatenate([w, c_ref[...]], axis=1)


def _build_g(z, c, psi):
    return pl.pallas_call(
        _tc_body,
        out_shape=jax.ShapeDtypeStruct((N, 2 * K), jnp.float32),
    )(z, c, psi)


def _sigmoid(x):
    return 1.0 / (1.0 + jnp.exp(-x))


def _sc_edge_scores(nch):
    mesh = plsc.VectorSubcoreMesh(core_axis_name="c", subcore_axis_name="s")
    e_pad = NW * nch * CH

    @functools.partial(
        pl.kernel,
        out_type=jax.ShapeDtypeStruct((e_pad,), jnp.float32),
        mesh=mesh,
        compiler_params=pltpu.CompilerParams(
            needs_layout_passes=False, use_tc_tiling_on_sc=False),
        scratch_types=[
            pltpu.VMEM((nch, CH), jnp.int32),
            pltpu.VMEM((nch, CH), jnp.int32),
            pltpu.VMEM((CH, 2 * K), jnp.float32),
            pltpu.VMEM((CH, 2 * K), jnp.float32),
            pltpu.VMEM((CH, 2 * K), jnp.float32),
            pltpu.VMEM((CH, 2 * K), jnp.float32),
            pltpu.VMEM((nch * CH,), jnp.float32),
            pltpu.VMEM_SHARED((N, 2 * K), jnp.float32),
            pltpu.SemaphoreType.DMA,
            pltpu.SemaphoreType.DMA,
        ],
    )
    def k(g_hbm, row_hbm, col_hbm, out_hbm,
          idx_r, idx_c, gr0, gr1, gc0, gc1, obf, spg, sem0, sem1):
        nc = plsc.get_sparse_core_info().num_cores
        wid = lax.axis_index("s") * nc + lax.axis_index("c")
        crow = wid * nch

        @pl.when(lax.axis_index("s") == 0)
        def _():
            pltpu.sync_copy(g_hbm, spg)
        pltpu.sync_copy(row_hbm.at[pl.ds(crow, nch)], idx_r)
        pltpu.sync_copy(col_hbm.at[pl.ds(crow, nch)], idx_c)
        plsc.subcore_barrier()

        slots = ((gr0, gc0, sem0), (gr1, gc1, sem1))

        def start(i, b):
            grb, gcb, semb = slots[b]
            pltpu.async_copy(spg.at[idx_r.at[i]], grb, semb)
            pltpu.async_copy(spg.at[idx_c.at[i]], gcb, semb)

        def wait(b):
            grb, gcb, semb = slots[b]
            pltpu.make_async_copy(g_hbm.at[pl.ds(0, CH)], grb, semb).wait()
            pltpu.make_async_copy(g_hbm.at[pl.ds(0, CH)], gcb, semb).wait()

        lane = lax.iota(jnp.int32, 16)

        def compute(i, b):
            grb, gcb, _ = slots[b]

            def group(t, _):
                za = jnp.zeros((16,), jnp.float32)
                r1 = za
                r2 = za
                for j in range(16):
                    e = t * 16 + j
                    wr = grb[e, pl.ds(0, K)]
                    cr = grb[e, pl.ds(K, K)]
                    wc = gcb[e, pl.ds(0, K)]
                    cc = gcb[e, pl.ds(K, K)]
                    s1 = jnp.sum(wr * cc)
                    s2 = jnp.sum(wc * cr)
                    m = lane == j
                    r1 = jnp.where(m, jnp.full((16,), s1), r1)
                    r2 = jnp.where(m, jnp.full((16,), s2), r2)
                s = 0.5 * (_sigmoid(r1) + _sigmoid(r2))
                obf[pl.ds(i * CH + t * 16, 16)] = s
                return 0

            lax.fori_loop(0, CH // 16, group, 0)

        start(0, 0)
        start(1, 1)

        def ring(p, _):
            for b in range(NBUF):
                i = p * NBUF + b
                wait(b)
                compute(i, b)

                @pl.when(i + NBUF < nch)
                def _():
                    start(i + NBUF, b)
            return 0

        lax.fori_loop(0, nch // NBUF, ring, 0)
        pltpu.sync_copy(obf, out_hbm.at[pl.ds(crow * CH, nch * CH)])

    return k


def kernel(z, edge_index, c, psi, mp_samples):
    del mp_samples
    e = edge_index.shape[1]
    nch = -(-e // (NW * CH * NBUF)) * NBUF
    e_pad = NW * nch * CH
    row = jnp.pad(edge_index[0].astype(jnp.int32), (0, e_pad - e))
    col = jnp.pad(edge_index[1].astype(jnp.int32), (0, e_pad - e))
    row = row.reshape(NW * nch, CH)
    col = col.reshape(NW * nch, CH)
    g = _build_g(z, c, psi)
    out = _sc_edge_scores(nch)(g, row, col)
    return out[:e]
